# baseline (speedup 1.0000x reference)
import jax
import jax.numpy as jnp
from jax import lax
from jax.experimental import pallas as pl
from jax.experimental.pallas import tpu as pltpu
from jax.experimental.pallas import tpu_sc as plsc

N = 10000
E = 320000
R = 8
D = 128

NC = 1
NS = 16
NW = NC * NS

EPT = E // NW
CH = 120
EPT_PAD = 20160
CHUNKS = EPT_PAD // CH
SLAB = 8
NSLAB = CHUNKS // SLAB
NBUF = 3

NPAD = 10112
ROWS_PER_SUB = NPAD // NS

BN = 1264
GRID = NPAD // BN


def _idx_body(src_ref, et_ref, o_ref):
    o_ref[...] = et_ref[...] * NPAD + src_ref[...]


def _edge_row_index(src, edge_type):
    f = pl.pallas_call(
        _idx_body,
        out_shape=jax.ShapeDtypeStruct((E // 128, 128), jnp.int32),
    )
    return f(src.reshape(E // 128, 128), edge_type.reshape(E // 128, 128))


def _proj_body(a_ref, w_ref, p_ref):
    h = jax.nn.relu(a_ref[...])
    y = jnp.dot(h, w_ref[...], preferred_element_type=jnp.float32)
    for r in range(R + 1):
        p_ref[r] = y[:, r * D:(r + 1) * D]


_proj = pl.pallas_call(
    _proj_body,
    grid=(GRID,),
    in_specs=[
        pl.BlockSpec((BN, D), lambda i: (i, 0)),
        pl.BlockSpec((D, (R + 1) * D), lambda i: (0, 0)),
    ],
    out_specs=pl.BlockSpec((R + 1, BN, D), lambda i: (0, i, 0)),
    out_shape=jax.ShapeDtypeStruct((R + 1, NPAD, D), jnp.float32),
)


FBN = 1000


def _final_body(a_ref, o_ref):
    o_ref[...] = jax.nn.relu(a_ref[...])


_final = pl.pallas_call(
    _final_body,
    grid=(N // FBN,),
    in_specs=[pl.BlockSpec((FBN, D), lambda i: (i, 0))],
    out_specs=pl.BlockSpec((FBN, D), lambda i: (i, 0)),
    out_shape=jax.ShapeDtypeStruct((N, D), jnp.float32),
)


def _sc_body(p_hbm, idx_hbm, dst_hbm, out_hbm,
             idx_s, dst_s, rows_v, acc_sh, gsem, ssem, isem, vsem):
    sid = lax.axis_index("s")
    r0 = sid * ROWS_PER_SUB
    init_d = pltpu.async_copy(p_hbm.at[pl.ds(R * NPAD + r0, ROWS_PER_SUB)],
                              acc_sh.at[pl.ds(r0, ROWS_PER_SUB)], vsem)
    pltpu.sync_copy(idx_hbm.at[sid, pl.ds(0, SLAB)], idx_s.at[0])
    pltpu.sync_copy(dst_hbm.at[sid, pl.ds(0, SLAB)], dst_s.at[0])
    pltpu.async_copy(idx_hbm.at[sid, pl.ds(SLAB, SLAB)], idx_s.at[1], isem)
    pltpu.async_copy(dst_hbm.at[sid, pl.ds(SLAB, SLAB)], dst_s.at[1], isem)

    def prologue(c, carry):
        b = lax.rem(c, NBUF)
        pltpu.async_copy(p_hbm.at[idx_s.at[0, c]], rows_v.at[b], gsem.at[b])
        return carry

    lax.fori_loop(0, NBUF - 1, prologue, 0)
    init_d.wait()
    plsc.subcore_barrier()

    def step(c, carry):
        b = lax.rem(c, NBUF)
        par = lax.rem(c // SLAB, 2)
        row = lax.rem(c, SLAB)
        pltpu.make_async_copy(p_hbm.at[idx_s.at[par, row]],
                              rows_v.at[b], gsem.at[b]).wait()
        pltpu.async_copy(rows_v.at[b], acc_sh.at[dst_s.at[par, row]],
                         ssem.at[b], add=True)

        @pl.when(c >= 1)
        def _():
            bprev = lax.rem(c - 1 + NBUF, NBUF)
            pltpu.make_async_copy(rows_v.at[bprev],
                                  acc_sh.at[dst_s.at[0, 0]],
                                  ssem.at[bprev]).wait()

        @pl.when((row == 0) & (c >= SLAB) & (c + 2 * SLAB <= CHUNKS))
        def _():
            nxt = pl.multiple_of(c + SLAB, SLAB)
            npar = lax.rem((c + SLAB) // SLAB, 2)
            pltpu.async_copy(idx_hbm.at[sid, pl.ds(nxt, SLAB)],
                             idx_s.at[npar], isem)
            pltpu.async_copy(dst_hbm.at[sid, pl.ds(nxt, SLAB)],
                             dst_s.at[npar], isem)

        c2 = c + NBUF - 1
        @pl.when((lax.rem(c2, SLAB) == 0) & (c2 < CHUNKS))
        def _():
            pltpu.make_async_copy(idx_hbm.at[sid, pl.ds(0, SLAB)],
                                  idx_s.at[0], isem).wait()
            pltpu.make_async_copy(dst_hbm.at[sid, pl.ds(0, SLAB)],
                                  dst_s.at[0], isem).wait()

        @pl.when(c2 < CHUNKS)
        def _():
            b2 = lax.rem(c2, NBUF)
            par2 = lax.rem(c2 // SLAB, 2)
            row2 = lax.rem(c2, SLAB)
            pltpu.async_copy(p_hbm.at[idx_s.at[par2, row2]],
                             rows_v.at[b2], gsem.at[b2])

        return carry

    lax.fori_loop(0, CHUNKS, step, 0)
    pltpu.make_async_copy(rows_v.at[lax.rem(CHUNKS - 1, NBUF)],
                          acc_sh.at[dst_s.at[0, 0]],
                          ssem.at[lax.rem(CHUNKS - 1, NBUF)]).wait()
    plsc.subcore_barrier()
    pltpu.sync_copy(acc_sh.at[pl.ds(r0, ROWS_PER_SUB)],
                    out_hbm.at[pl.ds(r0, ROWS_PER_SUB)])


_sc_scatter = pl.kernel(
    _sc_body,
    out_type=jax.ShapeDtypeStruct((NPAD, D), jnp.float32),
    mesh=plsc.VectorSubcoreMesh(core_axis_name="c", subcore_axis_name="s",
                                num_cores=NC),
    scratch_types=[
        pltpu.VMEM((2, SLAB, CH), jnp.int32),
        pltpu.VMEM((2, SLAB, CH), jnp.int32),
        pltpu.VMEM((NBUF, CH, D), jnp.float32),
        pltpu.VMEM_SHARED((NPAD, D), jnp.float32),
        pltpu.SemaphoreType.DMA((NBUF,)),
        pltpu.SemaphoreType.DMA((NBUF,)),
        pltpu.SemaphoreType.DMA,
        pltpu.SemaphoreType.DMA,
    ],
)


def kernel(node_ids, edge_index, edge_type, embed,
           W_rel1, W_self1, W_rel2, W_self2, W_rel3, W_self3):
    del node_ids
    src = edge_index[0]
    dst = edge_index[1]
    npadrows = NPAD - N
    tpad = jnp.arange(NW * (EPT_PAD - EPT), dtype=jnp.int32)
    trash_idx = (N + tpad % npadrows
                 + NPAD * (tpad % R)).reshape(NW, EPT_PAD - EPT)
    trash_dst = (N + tpad % npadrows).reshape(NW, EPT_PAD - EPT)
    idx_t = _edge_row_index(src, edge_type).reshape(NW, EPT)
    idx3d = jnp.concatenate([idx_t, trash_idx], 1).reshape(NW, CHUNKS, CH)
    dst3d = jnp.concatenate([dst.reshape(NW, EPT), trash_dst],
                            1).reshape(NW, CHUNKS, CH)
    embed_pad = jnp.pad(embed, ((0, NPAD - N), (0, 0)))

    agg = embed_pad
    for w_rel, w_self in ((W_rel1, W_self1), (W_rel2, W_self2),
                          (W_rel3, W_self3)):
        w = jnp.concatenate([w_rel, w_self[None]], 0)
        w = w.transpose(1, 0, 2).reshape(D, (R + 1) * D)
        p = _proj(agg, w)
        agg = _sc_scatter(p.reshape((R + 1) * NPAD, D), idx3d, dst3d)
    return _final(agg)


# speedup vs baseline: 1.0011x; 1.0011x over previous
import jax
import jax.numpy as jnp
from jax import lax
from jax.experimental import pallas as pl
from jax.experimental.pallas import tpu as pltpu
from jax.experimental.pallas import tpu_sc as plsc

N = 10000
E = 320000
R = 8
D = 128

NC = 1
NS = 16
NW = NC * NS

EPT = E // NW
CH = 120
EPT_PAD = 20160
CHUNKS = EPT_PAD // CH
SLAB = 8
NSLAB = CHUNKS // SLAB
NBUF = 3

NPAD = 10112
ROWS_PER_SUB = NPAD // NS

BN = 2528
GRID = NPAD // BN


def _idx_body(src_ref, et_ref, o_ref):
    o_ref[...] = et_ref[...] * NPAD + src_ref[...]


def _edge_row_index(src, edge_type):
    f = pl.pallas_call(
        _idx_body,
        out_shape=jax.ShapeDtypeStruct((E // 128, 128), jnp.int32),
    )
    return f(src.reshape(E // 128, 128), edge_type.reshape(E // 128, 128))


def _proj_body(a_ref, w_ref, p_ref):
    h = jax.nn.relu(a_ref[...])
    y = jnp.dot(h, w_ref[...], preferred_element_type=jnp.float32)
    for r in range(R + 1):
        p_ref[r] = y[:, r * D:(r + 1) * D]


_proj = pl.pallas_call(
    _proj_body,
    grid=(GRID,),
    in_specs=[
        pl.BlockSpec((BN, D), lambda i: (i, 0)),
        pl.BlockSpec((D, (R + 1) * D), lambda i: (0, 0)),
    ],
    out_specs=pl.BlockSpec((R + 1, BN, D), lambda i: (0, i, 0)),
    out_shape=jax.ShapeDtypeStruct((R + 1, NPAD, D), jnp.float32),
)


FBN = 1000


def _final_body(a_ref, o_ref):
    o_ref[...] = jax.nn.relu(a_ref[...])


_final = pl.pallas_call(
    _final_body,
    grid=(N // FBN,),
    in_specs=[pl.BlockSpec((FBN, D), lambda i: (i, 0))],
    out_specs=pl.BlockSpec((FBN, D), lambda i: (i, 0)),
    out_shape=jax.ShapeDtypeStruct((N, D), jnp.float32),
)


def _sc_body(p_hbm, idx_hbm, dst_hbm, out_hbm,
             idx_s, dst_s, rows_v, acc_sh, gsem, ssem, isem, vsem):
    sid = lax.axis_index("s")
    r0 = sid * ROWS_PER_SUB
    init_d = pltpu.async_copy(p_hbm.at[pl.ds(R * NPAD + r0, ROWS_PER_SUB)],
                              acc_sh.at[pl.ds(r0, ROWS_PER_SUB)], vsem)
    pltpu.sync_copy(idx_hbm.at[sid, pl.ds(0, SLAB)], idx_s.at[0])
    pltpu.sync_copy(dst_hbm.at[sid, pl.ds(0, SLAB)], dst_s.at[0])
    pltpu.async_copy(idx_hbm.at[sid, pl.ds(SLAB, SLAB)], idx_s.at[1], isem)
    pltpu.async_copy(dst_hbm.at[sid, pl.ds(SLAB, SLAB)], dst_s.at[1], isem)

    def prologue(c, carry):
        b = lax.rem(c, NBUF)
        pltpu.async_copy(p_hbm.at[idx_s.at[0, c]], rows_v.at[b], gsem.at[b])
        return carry

    lax.fori_loop(0, NBUF - 1, prologue, 0)
    init_d.wait()
    plsc.subcore_barrier()

    def step(c, carry):
        b = lax.rem(c, NBUF)
        par = lax.rem(c // SLAB, 2)
        row = lax.rem(c, SLAB)
        pltpu.make_async_copy(p_hbm.at[idx_s.at[par, row]],
                              rows_v.at[b], gsem.at[b]).wait()
        pltpu.async_copy(rows_v.at[b], acc_sh.at[dst_s.at[par, row]],
                         ssem.at[b], add=True)

        @pl.when(c >= 1)
        def _():
            bprev = lax.rem(c - 1 + NBUF, NBUF)
            pltpu.make_async_copy(rows_v.at[bprev],
                                  acc_sh.at[dst_s.at[0, 0]],
                                  ssem.at[bprev]).wait()

        @pl.when((row == 0) & (c >= SLAB) & (c + 2 * SLAB <= CHUNKS))
        def _():
            nxt = pl.multiple_of(c + SLAB, SLAB)
            npar = lax.rem((c + SLAB) // SLAB, 2)
            pltpu.async_copy(idx_hbm.at[sid, pl.ds(nxt, SLAB)],
                             idx_s.at[npar], isem)
            pltpu.async_copy(dst_hbm.at[sid, pl.ds(nxt, SLAB)],
                             dst_s.at[npar], isem)

        c2 = c + NBUF - 1
        @pl.when((lax.rem(c2, SLAB) == 0) & (c2 < CHUNKS))
        def _():
            pltpu.make_async_copy(idx_hbm.at[sid, pl.ds(0, SLAB)],
                                  idx_s.at[0], isem).wait()
            pltpu.make_async_copy(dst_hbm.at[sid, pl.ds(0, SLAB)],
                                  dst_s.at[0], isem).wait()

        @pl.when(c2 < CHUNKS)
        def _():
            b2 = lax.rem(c2, NBUF)
            par2 = lax.rem(c2 // SLAB, 2)
            row2 = lax.rem(c2, SLAB)
            pltpu.async_copy(p_hbm.at[idx_s.at[par2, row2]],
                             rows_v.at[b2], gsem.at[b2])

        return carry

    lax.fori_loop(0, CHUNKS, step, 0)
    pltpu.make_async_copy(rows_v.at[lax.rem(CHUNKS - 1, NBUF)],
                          acc_sh.at[dst_s.at[0, 0]],
                          ssem.at[lax.rem(CHUNKS - 1, NBUF)]).wait()
    plsc.subcore_barrier()
    pltpu.sync_copy(acc_sh.at[pl.ds(r0, ROWS_PER_SUB)],
                    out_hbm.at[pl.ds(r0, ROWS_PER_SUB)])


_sc_scatter = pl.kernel(
    _sc_body,
    out_type=jax.ShapeDtypeStruct((NPAD, D), jnp.float32),
    mesh=plsc.VectorSubcoreMesh(core_axis_name="c", subcore_axis_name="s",
                                num_cores=NC),
    scratch_types=[
        pltpu.VMEM((2, SLAB, CH), jnp.int32),
        pltpu.VMEM((2, SLAB, CH), jnp.int32),
        pltpu.VMEM((NBUF, CH, D), jnp.float32),
        pltpu.VMEM_SHARED((NPAD, D), jnp.float32),
        pltpu.SemaphoreType.DMA((NBUF,)),
        pltpu.SemaphoreType.DMA((NBUF,)),
        pltpu.SemaphoreType.DMA,
        pltpu.SemaphoreType.DMA,
    ],
)


def kernel(node_ids, edge_index, edge_type, embed,
           W_rel1, W_self1, W_rel2, W_self2, W_rel3, W_self3):
    del node_ids
    src = edge_index[0]
    dst = edge_index[1]
    npadrows = NPAD - N
    tpad = jnp.arange(NW * (EPT_PAD - EPT), dtype=jnp.int32)
    trash_idx = (N + tpad % npadrows
                 + NPAD * (tpad % R)).reshape(NW, EPT_PAD - EPT)
    trash_dst = (N + tpad % npadrows).reshape(NW, EPT_PAD - EPT)
    idx_t = _edge_row_index(src, edge_type).reshape(NW, EPT)
    idx3d = jnp.concatenate([idx_t, trash_idx], 1).reshape(NW, CHUNKS, CH)
    dst3d = jnp.concatenate([dst.reshape(NW, EPT), trash_dst],
                            1).reshape(NW, CHUNKS, CH)
    embed_pad = jnp.pad(embed, ((0, NPAD - N), (0, 0)))

    agg = embed_pad
    for w_rel, w_self in ((W_rel1, W_self1), (W_rel2, W_self2),
                          (W_rel3, W_self3)):
        w = jnp.concatenate([w_rel, w_self[None]], 0)
        w = w.transpose(1, 0, 2).reshape(D, (R + 1) * D)
        p = _proj(agg, w)
        agg = _sc_scatter(p.reshape((R + 1) * NPAD, D), idx3d, dst3d)
    return _final(agg)
